# Initial kernel scaffold; baseline (speedup 1.0000x reference)
import jax
import jax.numpy as jnp
from jax.experimental import pallas as pl


def kernel(patch, pos_table):
    raise NotImplementedError("write your pallas kernel here")


# speedup vs baseline: 1.0579x; 1.0579x over previous
import jax
import jax.numpy as jnp
from jax.experimental import pallas as pl
from jax.experimental.pallas import tpu as pltpu

NUM_PATCHES = 1024
EMBED_DIM = 768
BATCH = 64


def _body(pos_ref, patch_ref, out_ref, tpos_ref):
    @pl.when(pl.program_id(0) == 0)
    def _init():
        tpos_ref[...] = pos_ref[...].T

    out_ref[...] = patch_ref[...] + tpos_ref[...][None, :, :]


def kernel(patch, pos_table):
    return pl.pallas_call(
        _body,
        grid=(BATCH,),
        in_specs=[
            pl.BlockSpec((NUM_PATCHES, EMBED_DIM), lambda b: (0, 0)),
            pl.BlockSpec((1, EMBED_DIM, NUM_PATCHES), lambda b: (b, 0, 0)),
        ],
        out_specs=pl.BlockSpec((1, EMBED_DIM, NUM_PATCHES), lambda b: (b, 0, 0)),
        out_shape=jax.ShapeDtypeStruct((BATCH, EMBED_DIM, NUM_PATCHES), patch.dtype),
        scratch_shapes=[pltpu.VMEM((EMBED_DIM, NUM_PATCHES), jnp.float32)],
        compiler_params=pltpu.CompilerParams(
            dimension_semantics=("arbitrary",),
        ),
    )(pos_table, patch)


# speedup vs baseline: 1.0924x; 1.0327x over previous
import jax
import jax.numpy as jnp
from jax.experimental import pallas as pl
from jax.experimental.pallas import tpu as pltpu

NUM_PATCHES = 1024
EMBED_DIM = 768
BATCH = 64


def _body(pos_ref, patch_ref, out_ref, tpos_ref):
    @pl.when(pl.program_id(0) == 0)
    def _init():
        tpos_ref[...] = pos_ref[...].T

    out_ref[...] = patch_ref[...] + tpos_ref[...][None, :, :]


BB = 2


def kernel(patch, pos_table):
    return pl.pallas_call(
        _body,
        grid=(BATCH // BB,),
        in_specs=[
            pl.BlockSpec((NUM_PATCHES, EMBED_DIM), lambda b: (0, 0)),
            pl.BlockSpec((BB, EMBED_DIM, NUM_PATCHES), lambda b: (b, 0, 0)),
        ],
        out_specs=pl.BlockSpec((BB, EMBED_DIM, NUM_PATCHES), lambda b: (b, 0, 0)),
        out_shape=jax.ShapeDtypeStruct((BATCH, EMBED_DIM, NUM_PATCHES), patch.dtype),
        scratch_shapes=[pltpu.VMEM((EMBED_DIM, NUM_PATCHES), jnp.float32)],
        compiler_params=pltpu.CompilerParams(
            dimension_semantics=("arbitrary",),
        ),
    )(pos_table, patch)


# speedup vs baseline: 1.1122x; 1.0181x over previous
import jax
import jax.numpy as jnp
from jax.experimental import pallas as pl
from jax.experimental.pallas import tpu as pltpu

NUM_PATCHES = 1024
EMBED_DIM = 768
BATCH = 64


def _body(pos_ref, patch_ref, out_ref, tpos_ref):
    @pl.when(pl.program_id(0) == 0)
    def _init():
        tpos_ref[...] = pos_ref[...].T

    out_ref[...] = patch_ref[...] + tpos_ref[...][None, :, :]


BB = 4


def kernel(patch, pos_table):
    return pl.pallas_call(
        _body,
        grid=(BATCH // BB,),
        in_specs=[
            pl.BlockSpec((NUM_PATCHES, EMBED_DIM), lambda b: (0, 0)),
            pl.BlockSpec((BB, EMBED_DIM, NUM_PATCHES), lambda b: (b, 0, 0)),
        ],
        out_specs=pl.BlockSpec((BB, EMBED_DIM, NUM_PATCHES), lambda b: (b, 0, 0)),
        out_shape=jax.ShapeDtypeStruct((BATCH, EMBED_DIM, NUM_PATCHES), patch.dtype),
        scratch_shapes=[pltpu.VMEM((EMBED_DIM, NUM_PATCHES), jnp.float32)],
        compiler_params=pltpu.CompilerParams(
            dimension_semantics=("arbitrary",),
        ),
    )(pos_table, patch)
